# baseline (speedup 1.0000x reference)
import functools

import numpy as np

import jax
import jax.numpy as jnp
from jax import lax
from jax.experimental import pallas as pl
from jax.experimental.pallas import tpu as pltpu
from jax.experimental.pallas import tpu_sc as plsc

_R, _V = 32, 1_000_000
_CH = 20_000
_NCH = _V // _CH
_U = 10
_L = 16


def _rotl(x, d):
    return (x << np.uint32(d)) | (x >> np.uint32(32 - d))


def _threefry2x32(k0, k1, x0, x1):
    ks0, ks1 = np.uint32(k0), np.uint32(k1)
    ks2 = np.uint32(ks0 ^ ks1 ^ np.uint32(0x1BD11BDA))
    ks = (ks0, ks1, ks2)
    rot_a = (13, 15, 26, 6)
    rot_b = (17, 29, 16, 24)
    x0 = x0 + ks0
    x1 = x1 + ks1
    for g in range(5):
        for r in rot_a if g % 2 == 0 else rot_b:
            x0 = x0 + x1
            x1 = _rotl(x1, r)
            x1 = x0 ^ x1
        x0 = x0 + ks[(g + 1) % 3]
        x1 = x1 + ks[(g + 2) % 3] + np.uint32(g + 1)
    return x0, x1


def _pert_table():
    n_total = _R * _V
    out = np.empty(n_total, np.float32)
    step = 1 << 22
    for s in range(0, n_total, step):
        n = min(step, n_total - s)
        o0, o1 = _threefry2x32(0, 1, np.zeros(n, np.uint32),
                               np.arange(s, s + n, dtype=np.uint32))
        bits = o0 ^ o1
        u = ((bits >> np.uint32(9)) | np.uint32(0x3F800000)).view(np.float32)
        u = u - np.float32(1.0)
        noise = (-np.log1p(-u.astype(np.float64))).astype(np.float32)
        out[s:s + n] = -np.log(noise.astype(np.float64) + 1e-10)
    return out.reshape(_R, _V)


_PERT = _pert_table()

_mesh = plsc.VectorSubcoreMesh(core_axis_name="c", subcore_axis_name="s")


def _lane_gather(x, i):
    dnums = lax.GatherDimensionNumbers(
        offset_dims=(), collapsed_slice_dims=(0,), start_index_map=(0,))
    return lax.gather(x, i[:, None], dnums, slice_sizes=(1,),
                      mode=lax.GatherScatterMode.PROMISE_IN_BOUNDS)


@functools.partial(
    pl.kernel,
    mesh=_mesh,
    out_type=jax.ShapeDtypeStruct((_R * _L,), jnp.int32),
    scratch_types=[
        pltpu.VMEM((_CH,), jnp.float32),
        pltpu.VMEM((_CH,), jnp.float32),
        pltpu.VMEM((_CH,), jnp.float32),
        pltpu.VMEM((_CH,), jnp.float32),
        pltpu.VMEM((_L,), jnp.float32),
        pltpu.VMEM((_L,), jnp.int32),
        pltpu.VMEM((_L,), jnp.float32),
        pltpu.VMEM((_L,), jnp.int32),
        pltpu.SemaphoreType.DMA,
        pltpu.SemaphoreType.DMA,
        pltpu.SemaphoreType.DMA,
        pltpu.SemaphoreType.DMA,
    ],
)
def _sc_sample(x_hbm, c_hbm, t_hbm, out_hbm,
               xb0, xb1, cb0, cb1, tbuf, obuf, mref, iref,
               sx0, sx1, sc0, sc1):
    wid = lax.axis_index("s") * 2 + lax.axis_index("c")
    xbufs, cbufs = (xb0, xb1), (cb0, cb1)
    xsems, csems = (sx0, sx1), (sc0, sc1)

    def chunk_copies(k, b):
        src = pl.ds(wid * _V + k * _CH, _CH)
        return (
            pltpu.make_async_copy(x_hbm.at[src], xbufs[b], xsems[b]),
            pltpu.make_async_copy(c_hbm.at[src], cbufs[b], csems[b]),
        )

    pltpu.sync_copy(t_hbm.at[pl.ds(wid * _L, _L)], tbuf)
    t = tbuf[...]
    lane = lax.broadcasted_iota(jnp.int32, (_L,), 0)
    mref[...] = jnp.full((_L,), -jnp.inf, jnp.float32)
    iref[...] = jnp.zeros((_L,), jnp.int32)

    for b in (0, 1):
        for cp in chunk_copies(b, b):
            cp.start()

    def outer(k0, _):
        for b in (0, 1):
            k = 2 * k0 + b
            for cp in chunk_copies(k, b):
                cp.wait()
            xb, cb = xbufs[b], cbufs[b]
            base = lane + k * _CH

            def inner(i, carry):
                m, idx = carry
                off = i * _L
                s = xb[pl.ds(off, _L)] + t * cb[pl.ds(off, _L)]
                p = s > m
                m = jnp.where(p, s, m)
                idx = jnp.where(p, base + off, idx)
                return m, idx

            m, idx = plsc.parallel_loop(
                0, _CH // _L, unroll=_U, carry=(mref[...], iref[...]))(inner)
            mref[...] = m
            iref[...] = idx

            @pl.when(k + 2 < _NCH)
            def _():
                for cp in chunk_copies(k + 2, b):
                    cp.start()
        return _

    lax.fori_loop(0, _NCH // 2, outer, None)

    m, idx = mref[...], iref[...]
    for sh in (1, 2, 4, 8):
        prm = jnp.bitwise_xor(lane, sh)
        mp = _lane_gather(m, prm)
        ip = _lane_gather(idx, prm)
        win = (mp > m) | ((mp == m) & (ip < idx))
        m = jnp.where(win, mp, m)
        idx = jnp.where(win, ip, idx)
    obuf[...] = idx
    pltpu.sync_copy(obuf, out_hbm.at[pl.ds(wid * _L, _L)])


def kernel(logits, temperatures):
    t_rows = jnp.broadcast_to(
        temperatures.astype(jnp.float32).reshape(_R, 1), (_R, _L)).reshape(-1)
    out = _sc_sample(logits.astype(jnp.float32).reshape(-1),
                     _PERT.reshape(-1), t_rows)
    return out.reshape(_R, _L)[:, 0]


# speedup vs baseline: 1.0431x; 1.0431x over previous
import functools

import numpy as np

import jax
import jax.numpy as jnp
from jax import lax
from jax.experimental import pallas as pl
from jax.experimental.pallas import tpu as pltpu
from jax.experimental.pallas import tpu_sc as plsc

_R, _V = 32, 1_000_000
_CH = 20_000
_NCH = _V // _CH
_U = 10
_L = 16


def _rotl(x, d):
    return (x << np.uint32(d)) | (x >> np.uint32(32 - d))


def _threefry2x32(k0, k1, x0, x1):
    ks0, ks1 = np.uint32(k0), np.uint32(k1)
    ks2 = np.uint32(ks0 ^ ks1 ^ np.uint32(0x1BD11BDA))
    ks = (ks0, ks1, ks2)
    rot_a = (13, 15, 26, 6)
    rot_b = (17, 29, 16, 24)
    x0 = x0 + ks0
    x1 = x1 + ks1
    for g in range(5):
        for r in rot_a if g % 2 == 0 else rot_b:
            x0 = x0 + x1
            x1 = _rotl(x1, r)
            x1 = x0 ^ x1
        x0 = x0 + ks[(g + 1) % 3]
        x1 = x1 + ks[(g + 2) % 3] + np.uint32(g + 1)
    return x0, x1


def _pert_table():
    n_total = _R * _V
    out = np.empty(n_total, np.float32)
    step = 1 << 22
    for s in range(0, n_total, step):
        n = min(step, n_total - s)
        o0, o1 = _threefry2x32(0, 1, np.zeros(n, np.uint32),
                               np.arange(s, s + n, dtype=np.uint32))
        bits = o0 ^ o1
        u = ((bits >> np.uint32(9)) | np.uint32(0x3F800000)).view(np.float32)
        u = u - np.float32(1.0)
        noise = (-np.log1p(-u.astype(np.float64))).astype(np.float32)
        out[s:s + n] = -np.log(noise.astype(np.float64) + 1e-10)
    return out.reshape(_R, _V)


_PERT = _pert_table()

_mesh = plsc.VectorSubcoreMesh(core_axis_name="c", subcore_axis_name="s")


def _lane_gather(x, i):
    dnums = lax.GatherDimensionNumbers(
        offset_dims=(), collapsed_slice_dims=(0,), start_index_map=(0,))
    return lax.gather(x, i[:, None], dnums, slice_sizes=(1,),
                      mode=lax.GatherScatterMode.PROMISE_IN_BOUNDS)


@functools.partial(
    pl.kernel,
    mesh=_mesh,
    out_type=jax.ShapeDtypeStruct((_R * _L,), jnp.int32),
    scratch_types=[
        pltpu.VMEM((_CH,), jnp.float32),
        pltpu.VMEM((_CH,), jnp.float32),
        pltpu.VMEM((_CH,), jnp.float32),
        pltpu.VMEM((_CH,), jnp.float32),
        pltpu.VMEM((_L,), jnp.float32),
        pltpu.VMEM((_L,), jnp.int32),
        pltpu.VMEM((_U * _L,), jnp.float32),
        pltpu.VMEM((_U * _L,), jnp.int32),
        pltpu.SemaphoreType.DMA,
        pltpu.SemaphoreType.DMA,
        pltpu.SemaphoreType.DMA,
        pltpu.SemaphoreType.DMA,
    ],
)
def _sc_sample(x_hbm, c_hbm, t_hbm, out_hbm,
               xb0, xb1, cb0, cb1, tbuf, obuf, mref, iref,
               sx0, sx1, sc0, sc1):
    wid = lax.axis_index("s") * 2 + lax.axis_index("c")
    xbufs, cbufs = (xb0, xb1), (cb0, cb1)
    xsems, csems = (sx0, sx1), (sc0, sc1)

    def chunk_copies(k, b):
        src = pl.ds(wid * _V + k * _CH, _CH)
        return (
            pltpu.make_async_copy(x_hbm.at[src], xbufs[b], xsems[b]),
            pltpu.make_async_copy(c_hbm.at[src], cbufs[b], csems[b]),
        )

    pltpu.sync_copy(t_hbm.at[pl.ds(wid * _L, _L)], tbuf)
    t = tbuf[...]
    lane = lax.broadcasted_iota(jnp.int32, (_L,), 0)
    for j in range(_U):
        mref[pl.ds(j * _L, _L)] = jnp.full((_L,), -jnp.inf, jnp.float32)
        iref[pl.ds(j * _L, _L)] = jnp.zeros((_L,), jnp.int32)

    for b in (0, 1):
        for cp in chunk_copies(b, b):
            cp.start()

    _W = _L * _U

    def outer(k0, _):
        for b in (0, 1):
            k = 2 * k0 + b
            for cp in chunk_copies(k, b):
                cp.wait()
            xb, cb = xbufs[b], cbufs[b]
            base = lane + k * _CH

            def inner(i, carry):
                ms, gs = carry[:_U], carry[_U:]
                off0 = i * _W
                out_m, out_g = [], []
                for j in range(_U):
                    off = off0 + j * _L
                    s = xb[pl.ds(off, _L)] + t * cb[pl.ds(off, _L)]
                    p = s > ms[j]
                    out_m.append(jnp.where(p, s, ms[j]))
                    out_g.append(jnp.where(p, base + off, gs[j]))
                return tuple(out_m) + tuple(out_g)

            carry0 = tuple(mref[pl.ds(j * _L, _L)] for j in range(_U)) + \
                     tuple(iref[pl.ds(j * _L, _L)] for j in range(_U))
            res = plsc.parallel_loop(
                0, _CH // _W, unroll=1, carry=carry0)(inner)
            for j in range(_U):
                mref[pl.ds(j * _L, _L)] = res[j]
                iref[pl.ds(j * _L, _L)] = res[_U + j]

            @pl.when(k + 2 < _NCH)
            def _():
                for cp in chunk_copies(k + 2, b):
                    cp.start()
        return _

    lax.fori_loop(0, _NCH // 2, outer, None)

    m = mref[pl.ds(0, _L)]
    idx = iref[pl.ds(0, _L)]
    for j in range(1, _U):
        mj = mref[pl.ds(j * _L, _L)]
        ij = iref[pl.ds(j * _L, _L)]
        win = (mj > m) | ((mj == m) & (ij < idx))
        m = jnp.where(win, mj, m)
        idx = jnp.where(win, ij, idx)
    for sh in (1, 2, 4, 8):
        prm = jnp.bitwise_xor(lane, sh)
        mp = _lane_gather(m, prm)
        ip = _lane_gather(idx, prm)
        win = (mp > m) | ((mp == m) & (ip < idx))
        m = jnp.where(win, mp, m)
        idx = jnp.where(win, ip, idx)
    obuf[...] = idx
    pltpu.sync_copy(obuf, out_hbm.at[pl.ds(wid * _L, _L)])


def kernel(logits, temperatures):
    t_rows = jnp.broadcast_to(
        temperatures.astype(jnp.float32).reshape(_R, 1), (_R, _L)).reshape(-1)
    out = _sc_sample(logits.astype(jnp.float32).reshape(-1),
                     _PERT.reshape(-1), t_rows)
    return out.reshape(_R, _L)[:, 0]


# speedup vs baseline: 1.0485x; 1.0052x over previous
import functools

import numpy as np

import jax
import jax.numpy as jnp
from jax import lax
from jax.experimental import pallas as pl
from jax.experimental.pallas import tpu as pltpu
from jax.experimental.pallas import tpu_sc as plsc

_R, _V = 32, 1_000_000
_CH = 10_000
_NCH = _V // _CH
_D = 4
_U = 5
_L = 16
_W = _L * _U


def _rotl(x, d):
    return (x << np.uint32(d)) | (x >> np.uint32(32 - d))


def _threefry2x32(k0, k1, x0, x1):
    ks0, ks1 = np.uint32(k0), np.uint32(k1)
    ks2 = np.uint32(ks0 ^ ks1 ^ np.uint32(0x1BD11BDA))
    ks = (ks0, ks1, ks2)
    rot_a = (13, 15, 26, 6)
    rot_b = (17, 29, 16, 24)
    x0 = x0 + ks0
    x1 = x1 + ks1
    for g in range(5):
        for r in rot_a if g % 2 == 0 else rot_b:
            x0 = x0 + x1
            x1 = _rotl(x1, r)
            x1 = x0 ^ x1
        x0 = x0 + ks[(g + 1) % 3]
        x1 = x1 + ks[(g + 2) % 3] + np.uint32(g + 1)
    return x0, x1


def _pert_table():
    n_total = _R * _V
    out = np.empty(n_total, np.float32)
    step = 1 << 22
    for s in range(0, n_total, step):
        n = min(step, n_total - s)
        o0, o1 = _threefry2x32(0, 1, np.zeros(n, np.uint32),
                               np.arange(s, s + n, dtype=np.uint32))
        bits = o0 ^ o1
        u = ((bits >> np.uint32(9)) | np.uint32(0x3F800000)).view(np.float32)
        u = u - np.float32(1.0)
        noise = (-np.log1p(-u.astype(np.float64))).astype(np.float32)
        out[s:s + n] = -np.log(noise.astype(np.float64) + 1e-10)
    return out.reshape(_R, _V)


_PERT = _pert_table()

_mesh = plsc.VectorSubcoreMesh(core_axis_name="c", subcore_axis_name="s")


def _lane_gather(x, i):
    dnums = lax.GatherDimensionNumbers(
        offset_dims=(), collapsed_slice_dims=(0,), start_index_map=(0,))
    return lax.gather(x, i[:, None], dnums, slice_sizes=(1,),
                      mode=lax.GatherScatterMode.PROMISE_IN_BOUNDS)


@functools.partial(
    pl.kernel,
    mesh=_mesh,
    out_type=jax.ShapeDtypeStruct((_R * _L,), jnp.int32),
    scratch_types=(
        [pltpu.VMEM((_CH,), jnp.float32) for _ in range(2 * _D)]
        + [
            pltpu.VMEM((_L,), jnp.float32),
            pltpu.VMEM((_L,), jnp.int32),
            pltpu.VMEM((_U * _L,), jnp.float32),
            pltpu.VMEM((_U * _L,), jnp.int32),
        ]
        + [pltpu.SemaphoreType.DMA for _ in range(2 * _D)]
    ),
)
def _sc_sample(x_hbm, c_hbm, t_hbm, out_hbm, *refs):
    xbufs = refs[0:_D]
    cbufs = refs[_D:2 * _D]
    tbuf, obuf, mref, iref = refs[2 * _D:2 * _D + 4]
    xsems = refs[2 * _D + 4:3 * _D + 4]
    csems = refs[3 * _D + 4:4 * _D + 4]

    wid = lax.axis_index("s") * 2 + lax.axis_index("c")

    def chunk_copies(k, b):
        src = pl.ds(wid * _V + k * _CH, _CH)
        return (
            pltpu.make_async_copy(x_hbm.at[src], xbufs[b], xsems[b]),
            pltpu.make_async_copy(c_hbm.at[src], cbufs[b], csems[b]),
        )

    pltpu.sync_copy(t_hbm.at[pl.ds(wid * _L, _L)], tbuf)
    t = tbuf[...]
    lane = lax.broadcasted_iota(jnp.int32, (_L,), 0)
    for j in range(_U):
        mref[pl.ds(j * _L, _L)] = jnp.full((_L,), -jnp.inf, jnp.float32)
        iref[pl.ds(j * _L, _L)] = jnp.zeros((_L,), jnp.int32)

    for b in range(_D):
        for cp in chunk_copies(b, b):
            cp.start()

    def outer(k0, _):
        for b in range(_D):
            k = _D * k0 + b
            for cp in chunk_copies(k, b):
                cp.wait()
            xb, cb = xbufs[b], cbufs[b]
            base = lane + k * _CH

            def inner(i, carry):
                ms, gs = carry[:_U], carry[_U:]
                off0 = i * _W
                out_m, out_g = [], []
                for j in range(_U):
                    off = off0 + j * _L
                    s = xb[pl.ds(off, _L)] + t * cb[pl.ds(off, _L)]
                    p = s > ms[j]
                    out_m.append(jnp.where(p, s, ms[j]))
                    out_g.append(jnp.where(p, base + off, gs[j]))
                return tuple(out_m) + tuple(out_g)

            carry0 = tuple(mref[pl.ds(j * _L, _L)] for j in range(_U)) + \
                     tuple(iref[pl.ds(j * _L, _L)] for j in range(_U))
            res = plsc.parallel_loop(
                0, _CH // _W, unroll=2, carry=carry0)(inner)
            for j in range(_U):
                mref[pl.ds(j * _L, _L)] = res[j]
                iref[pl.ds(j * _L, _L)] = res[_U + j]

            @pl.when(k + _D < _NCH)
            def _():
                for cp in chunk_copies(k + _D, b):
                    cp.start()
        return _

    lax.fori_loop(0, _NCH // _D, outer, None)

    m = mref[pl.ds(0, _L)]
    idx = iref[pl.ds(0, _L)]
    for j in range(1, _U):
        mj = mref[pl.ds(j * _L, _L)]
        ij = iref[pl.ds(j * _L, _L)]
        win = (mj > m) | ((mj == m) & (ij < idx))
        m = jnp.where(win, mj, m)
        idx = jnp.where(win, ij, idx)
    for sh in (1, 2, 4, 8):
        prm = jnp.bitwise_xor(lane, sh)
        mp = _lane_gather(m, prm)
        ip = _lane_gather(idx, prm)
        win = (mp > m) | ((mp == m) & (ip < idx))
        m = jnp.where(win, mp, m)
        idx = jnp.where(win, ip, idx)
    obuf[...] = idx
    pltpu.sync_copy(obuf, out_hbm.at[pl.ds(wid * _L, _L)])


def kernel(logits, temperatures):
    t_rows = jnp.broadcast_to(
        temperatures.astype(jnp.float32).reshape(_R, 1), (_R, _L)).reshape(-1)
    out = _sc_sample(logits.astype(jnp.float32).reshape(-1),
                     _PERT.reshape(-1), t_rows)
    return out.reshape(_R, _L)[:, 0]


# speedup vs baseline: 1.0517x; 1.0031x over previous
import functools

import numpy as np

import jax
import jax.numpy as jnp
from jax import lax
from jax.experimental import pallas as pl
from jax.experimental.pallas import tpu as pltpu
from jax.experimental.pallas import tpu_sc as plsc

_R, _V = 32, 1_000_000
_B = 65_536
_NBLK = 15
_VTC = _NBLK * _B
_VSC = _V - _VTC
_CH = 4_240
_D = 4
_U = 5
_L = 16
_W = _L * _U


def _rotl(x, d):
    return (x << np.uint32(d)) | (x >> np.uint32(32 - d))


def _threefry2x32(k0, k1, x0, x1):
    ks0, ks1 = np.uint32(k0), np.uint32(k1)
    ks2 = np.uint32(ks0 ^ ks1 ^ np.uint32(0x1BD11BDA))
    ks = (ks0, ks1, ks2)
    rot_a = (13, 15, 26, 6)
    rot_b = (17, 29, 16, 24)
    x0 = x0 + ks0
    x1 = x1 + ks1
    for g in range(5):
        for r in rot_a if g % 2 == 0 else rot_b:
            x0 = x0 + x1
            x1 = _rotl(x1, r)
            x1 = x0 ^ x1
        x0 = x0 + ks[(g + 1) % 3]
        x1 = x1 + ks[(g + 2) % 3] + np.uint32(g + 1)
    return x0, x1


def _pert_table():
    n_total = _R * _V
    out = np.empty(n_total, np.float32)
    step = 1 << 22
    for s in range(0, n_total, step):
        n = min(step, n_total - s)
        o0, o1 = _threefry2x32(0, 1, np.zeros(n, np.uint32),
                               np.arange(s, s + n, dtype=np.uint32))
        bits = o0 ^ o1
        u = ((bits >> np.uint32(9)) | np.uint32(0x3F800000)).view(np.float32)
        u = u - np.float32(1.0)
        noise = (-np.log1p(-u.astype(np.float64))).astype(np.float32)
        out[s:s + n] = -np.log(noise.astype(np.float64) + 1e-10)
    return out.reshape(_R, _V)


_PERT = _pert_table()


def _tc_body(t_ref, x_ref, c_ref, oi_ref, om_ref, m_ref, i_ref):
    pid = pl.program_id(0)

    @pl.when(pid == 0)
    def _():
        m_ref[...] = jnp.full_like(m_ref[...], -jnp.inf)
        i_ref[...] = jnp.zeros_like(i_ref[...])

    s = x_ref[...] + t_ref[...] * c_ref[...]
    m = jnp.max(s, axis=1, keepdims=True)
    a = (jnp.argmax(s, axis=1).astype(jnp.int32) + pid * _B).reshape(_R, 1)
    better = m > m_ref[...]
    i_ref[...] = jnp.where(better, a, i_ref[...])
    m_ref[...] = jnp.where(better, m, m_ref[...])

    @pl.when(pid == _NBLK - 1)
    def _():
        oi_ref[...] = i_ref[...]
        om_ref[...] = m_ref[...]


def _tc_sample(t2, x, c):
    return pl.pallas_call(
        _tc_body,
        grid=(_NBLK,),
        in_specs=[
            pl.BlockSpec((_R, 1), lambda i: (0, 0)),
            pl.BlockSpec((_R, _B), lambda i: (0, i)),
            pl.BlockSpec((_R, _B), lambda i: (0, i)),
        ],
        out_specs=[
            pl.BlockSpec((_R, 1), lambda i: (0, 0)),
            pl.BlockSpec((_R, 1), lambda i: (0, 0)),
        ],
        out_shape=[
            jax.ShapeDtypeStruct((_R, 1), jnp.int32),
            jax.ShapeDtypeStruct((_R, 1), jnp.float32),
        ],
        scratch_shapes=[
            pltpu.VMEM((_R, 1), jnp.float32),
            pltpu.VMEM((_R, 1), jnp.int32),
        ],
        compiler_params=pltpu.CompilerParams(
            dimension_semantics=("arbitrary",),
        ),
    )(t2, x, c)


_mesh = plsc.VectorSubcoreMesh(core_axis_name="c", subcore_axis_name="s")


def _lane_gather(x, i):
    dnums = lax.GatherDimensionNumbers(
        offset_dims=(), collapsed_slice_dims=(0,), start_index_map=(0,))
    return lax.gather(x, i[:, None], dnums, slice_sizes=(1,),
                      mode=lax.GatherScatterMode.PROMISE_IN_BOUNDS)


@functools.partial(
    pl.kernel,
    mesh=_mesh,
    out_type=(
        jax.ShapeDtypeStruct((_R * _L,), jnp.int32),
        jax.ShapeDtypeStruct((_R * _L,), jnp.float32),
    ),
    scratch_types=(
        [pltpu.VMEM((_CH,), jnp.float32) for _ in range(2 * _D)]
        + [
            pltpu.VMEM((_L,), jnp.float32),
            pltpu.VMEM((_L,), jnp.int32),
            pltpu.VMEM((_L,), jnp.float32),
        ]
        + [pltpu.SemaphoreType.DMA for _ in range(2 * _D)]
    ),
)
def _sc_sample(x_hbm, c_hbm, t_hbm, oi_hbm, om_hbm, *refs):
    xbufs = refs[0:_D]
    cbufs = refs[_D:2 * _D]
    tbuf, obuf, mbuf = refs[2 * _D:2 * _D + 3]
    xsems = refs[2 * _D + 3:3 * _D + 3]
    csems = refs[3 * _D + 3:4 * _D + 3]

    wid = lax.axis_index("s") * 2 + lax.axis_index("c")

    def chunk_copies(k, b):
        src = pl.ds(wid * _V + _VTC + k * _CH, _CH)
        return (
            pltpu.make_async_copy(x_hbm.at[src], xbufs[b], xsems[b]),
            pltpu.make_async_copy(c_hbm.at[src], cbufs[b], csems[b]),
        )

    pltpu.sync_copy(t_hbm.at[pl.ds(wid * _L, _L)], tbuf)
    t = tbuf[...]
    lane = lax.broadcasted_iota(jnp.int32, (_L,), 0)

    for b in range(_D):
        for cp in chunk_copies(b, b):
            cp.start()

    ms = [jnp.full((_L,), -jnp.inf, jnp.float32) for _ in range(_U)]
    gs = [jnp.zeros((_L,), jnp.int32) for _ in range(_U)]
    for b in range(_D):
        for cp in chunk_copies(b, b):
            cp.wait()
        xb, cb = xbufs[b], cbufs[b]
        base = lane + _VTC + b * _CH

        def inner(i, carry, xb=xb, cb=cb, base=base):
            cms, cgs = carry[:_U], carry[_U:]
            off0 = i * _W
            out_m, out_g = [], []
            for j in range(_U):
                off = off0 + j * _L
                s = xb[pl.ds(off, _L)] + t * cb[pl.ds(off, _L)]
                p = s > cms[j]
                out_m.append(jnp.where(p, s, cms[j]))
                out_g.append(jnp.where(p, base + off, cgs[j]))
            return tuple(out_m) + tuple(out_g)

        res = plsc.parallel_loop(
            0, _CH // _W, unroll=2, carry=tuple(ms) + tuple(gs))(inner)
        ms, gs = list(res[:_U]), list(res[_U:])

    m, idx = ms[0], gs[0]
    for j in range(1, _U):
        win = (ms[j] > m) | ((ms[j] == m) & (gs[j] < idx))
        m = jnp.where(win, ms[j], m)
        idx = jnp.where(win, gs[j], idx)
    for sh in (1, 2, 4, 8):
        prm = jnp.bitwise_xor(lane, sh)
        mp = _lane_gather(m, prm)
        ip = _lane_gather(idx, prm)
        win = (mp > m) | ((mp == m) & (ip < idx))
        m = jnp.where(win, mp, m)
        idx = jnp.where(win, ip, idx)
    obuf[...] = idx
    mbuf[...] = m
    pltpu.sync_copy(obuf, oi_hbm.at[pl.ds(wid * _L, _L)])
    pltpu.sync_copy(mbuf, om_hbm.at[pl.ds(wid * _L, _L)])


_PERT_J = jnp.asarray(_PERT)


def kernel(logits, temperatures):
    x = logits.astype(jnp.float32)
    t2 = temperatures.astype(jnp.float32).reshape(_R, 1)
    t_rows = jnp.broadcast_to(t2, (_R, _L)).reshape(-1)
    sc_idx, sc_max = _sc_sample(x.reshape(-1), _PERT_J.reshape(-1), t_rows)
    tc_idx, tc_max = _tc_sample(t2, x, _PERT_J)
    sc_idx = sc_idx.reshape(_R, _L)[:, 0]
    sc_max = sc_max.reshape(_R, _L)[:, 0]
    tc_idx = tc_idx.reshape(_R)
    tc_max = tc_max.reshape(_R)
    return jnp.where(sc_max > tc_max, sc_idx, tc_idx)


# speedup vs baseline: 27.3847x; 26.0383x over previous
import functools

import numpy as np

import jax
import jax.numpy as jnp
from jax import lax
from jax.experimental import pallas as pl
from jax.experimental.pallas import tpu as pltpu
from jax.experimental.pallas import tpu_sc as plsc

_R, _V = 32, 1_000_000
_B = 65_536
_NBLK = 15
_VTC = _NBLK * _B
_VSC = _V - _VTC
_CH = 4_240
_D = 4
_U = 5
_L = 16
_W = _L * _U


def _rotl(x, d):
    return (x << np.uint32(d)) | (x >> np.uint32(32 - d))


def _threefry2x32(k0, k1, x0, x1):
    ks0, ks1 = np.uint32(k0), np.uint32(k1)
    ks2 = np.uint32(ks0 ^ ks1 ^ np.uint32(0x1BD11BDA))
    ks = (ks0, ks1, ks2)
    rot_a = (13, 15, 26, 6)
    rot_b = (17, 29, 16, 24)
    x0 = x0 + ks0
    x1 = x1 + ks1
    for g in range(5):
        for r in rot_a if g % 2 == 0 else rot_b:
            x0 = x0 + x1
            x1 = _rotl(x1, r)
            x1 = x0 ^ x1
        x0 = x0 + ks[(g + 1) % 3]
        x1 = x1 + ks[(g + 2) % 3] + np.uint32(g + 1)
    return x0, x1


def _pert_table():
    n_total = _R * _V
    out = np.empty(n_total, np.float32)
    step = 1 << 22
    for s in range(0, n_total, step):
        n = min(step, n_total - s)
        o0, o1 = _threefry2x32(0, 1, np.zeros(n, np.uint32),
                               np.arange(s, s + n, dtype=np.uint32))
        bits = o0 ^ o1
        u = ((bits >> np.uint32(9)) | np.uint32(0x3F800000)).view(np.float32)
        u = u - np.float32(1.0)
        noise = (-np.log1p(-u.astype(np.float64))).astype(np.float32)
        out[s:s + n] = -np.log(noise.astype(np.float64) + 1e-10)
    return out.reshape(_R, _V)


_PERT = _pert_table()


def _tc_body(t_ref, x_ref, c_ref, oi_ref, om_ref, m_ref, i_ref):
    pid = pl.program_id(0)

    @pl.when(pid == 0)
    def _():
        m_ref[...] = jnp.full_like(m_ref[...], -jnp.inf)
        i_ref[...] = jnp.zeros_like(i_ref[...])

    s = x_ref[...] + t_ref[...] * c_ref[...]
    m = jnp.max(s, axis=1, keepdims=True)
    a = (jnp.argmax(s, axis=1).astype(jnp.int32) + pid * _B).reshape(_R, 1)
    better = m > m_ref[...]
    i_ref[...] = jnp.where(better, a, i_ref[...])
    m_ref[...] = jnp.where(better, m, m_ref[...])

    @pl.when(pid == _NBLK - 1)
    def _():
        oi_ref[...] = i_ref[...]
        om_ref[...] = m_ref[...]


def _tc_sample(t2, x, c):
    return pl.pallas_call(
        _tc_body,
        grid=(_NBLK,),
        in_specs=[
            pl.BlockSpec((_R, 1), lambda i: (0, 0)),
            pl.BlockSpec((_R, _B), lambda i: (0, i)),
            pl.BlockSpec((_R, _B), lambda i: (0, i)),
        ],
        out_specs=[
            pl.BlockSpec((_R, 1), lambda i: (0, 0)),
            pl.BlockSpec((_R, 1), lambda i: (0, 0)),
        ],
        out_shape=[
            jax.ShapeDtypeStruct((_R, 1), jnp.int32),
            jax.ShapeDtypeStruct((_R, 1), jnp.float32),
        ],
        scratch_shapes=[
            pltpu.VMEM((_R, 1), jnp.float32),
            pltpu.VMEM((_R, 1), jnp.int32),
        ],
        compiler_params=pltpu.CompilerParams(
            dimension_semantics=("arbitrary",),
        ),
    )(t2, x, c)


_mesh = plsc.VectorSubcoreMesh(core_axis_name="c", subcore_axis_name="s")


def _lane_gather(x, i):
    dnums = lax.GatherDimensionNumbers(
        offset_dims=(), collapsed_slice_dims=(0,), start_index_map=(0,))
    return lax.gather(x, i[:, None], dnums, slice_sizes=(1,),
                      mode=lax.GatherScatterMode.PROMISE_IN_BOUNDS)


@functools.partial(
    pl.kernel,
    mesh=_mesh,
    out_type=(
        jax.ShapeDtypeStruct((_R * _L,), jnp.int32),
        jax.ShapeDtypeStruct((_R * _L,), jnp.float32),
    ),
    scratch_types=(
        [pltpu.VMEM((_CH,), jnp.float32) for _ in range(2 * _D)]
        + [
            pltpu.VMEM((_L,), jnp.float32),
            pltpu.VMEM((_L,), jnp.int32),
            pltpu.VMEM((_L,), jnp.float32),
        ]
        + [pltpu.SemaphoreType.DMA for _ in range(2 * _D)]
    ),
)
def _sc_sample(x_hbm, c_hbm, t_hbm, oi_hbm, om_hbm, *refs):
    xbufs = refs[0:_D]
    cbufs = refs[_D:2 * _D]
    tbuf, obuf, mbuf = refs[2 * _D:2 * _D + 3]
    xsems = refs[2 * _D + 3:3 * _D + 3]
    csems = refs[3 * _D + 3:4 * _D + 3]

    wid = lax.axis_index("s") * 2 + lax.axis_index("c")

    def chunk_copies(k, b):
        src = pl.ds(wid * _VSC + k * _CH, _CH)
        return (
            pltpu.make_async_copy(x_hbm.at[src], xbufs[b], xsems[b]),
            pltpu.make_async_copy(c_hbm.at[src], cbufs[b], csems[b]),
        )

    pltpu.sync_copy(t_hbm.at[pl.ds(wid * _L, _L)], tbuf)
    t = tbuf[...]
    lane = lax.broadcasted_iota(jnp.int32, (_L,), 0)

    for b in range(_D):
        for cp in chunk_copies(b, b):
            cp.start()

    ms = [jnp.full((_L,), -jnp.inf, jnp.float32) for _ in range(_U)]
    gs = [jnp.zeros((_L,), jnp.int32) for _ in range(_U)]
    for b in range(_D):
        for cp in chunk_copies(b, b):
            cp.wait()
        xb, cb = xbufs[b], cbufs[b]
        base = lane + _VTC + b * _CH

        def inner(i, carry, xb=xb, cb=cb, base=base):
            cms, cgs = carry[:_U], carry[_U:]
            off0 = i * _W
            out_m, out_g = [], []
            for j in range(_U):
                off = off0 + j * _L
                s = xb[pl.ds(off, _L)] + t * cb[pl.ds(off, _L)]
                p = s > cms[j]
                out_m.append(jnp.where(p, s, cms[j]))
                out_g.append(jnp.where(p, base + off, cgs[j]))
            return tuple(out_m) + tuple(out_g)

        res = plsc.parallel_loop(
            0, _CH // _W, unroll=2, carry=tuple(ms) + tuple(gs))(inner)
        ms, gs = list(res[:_U]), list(res[_U:])

    m, idx = ms[0], gs[0]
    for j in range(1, _U):
        win = (ms[j] > m) | ((ms[j] == m) & (gs[j] < idx))
        m = jnp.where(win, ms[j], m)
        idx = jnp.where(win, gs[j], idx)
    for sh in (1, 2, 4, 8):
        prm = jnp.bitwise_xor(lane, sh)
        mp = _lane_gather(m, prm)
        ip = _lane_gather(idx, prm)
        win = (mp > m) | ((mp == m) & (ip < idx))
        m = jnp.where(win, mp, m)
        idx = jnp.where(win, ip, idx)
    obuf[...] = idx
    mbuf[...] = m
    pltpu.sync_copy(obuf, oi_hbm.at[pl.ds(wid * _L, _L)])
    pltpu.sync_copy(mbuf, om_hbm.at[pl.ds(wid * _L, _L)])


_PERT_J = jnp.asarray(_PERT)
_PERT_SC = jnp.asarray(np.ascontiguousarray(_PERT[:, _VTC:]).reshape(-1))


def kernel(logits, temperatures):
    x = logits.astype(jnp.float32)
    t2 = temperatures.astype(jnp.float32).reshape(_R, 1)
    t_rows = jnp.broadcast_to(t2, (_R, _L)).reshape(-1)
    x_sc = x[:, _VTC:].reshape(-1)
    sc_idx, sc_max = _sc_sample(x_sc, _PERT_SC, t_rows)
    tc_idx, tc_max = _tc_sample(t2, x, _PERT_J)
    sc_idx = sc_idx.reshape(_R, _L)[:, 0]
    sc_max = sc_max.reshape(_R, _L)[:, 0]
    tc_idx = tc_idx.reshape(_R)
    tc_max = tc_max.reshape(_R)
    return jnp.where(sc_max > tc_max, sc_idx, tc_idx)
